# Initial kernel scaffold; baseline (speedup 1.0000x reference)
import jax
import jax.numpy as jnp
from jax.experimental import pallas as pl


def kernel(input, level_weight, channel_weight, centroid_weight):
    raise NotImplementedError("write your pallas kernel here")


# speedup vs baseline: 1.9837x; 1.9837x over previous
import functools

import jax
import jax.numpy as jnp
from jax import lax
from jax.experimental import pallas as pl
from jax.experimental.pallas import tpu as pltpu
from jax.experimental.pallas import tpu_sc as plsc

NUM_LEVELS = 201
N_GRAM_SIZE = 4
LOW, HIGH = -100.0, 100.0

_SC_CORES = 2
_SC_SUBCORES = 16
_NW = _SC_CORES * _SC_SUBCORES


def _prep_kernel(inp_ref, lw_ref, cw_ref, idx_ref, bt_ref):
    x = inp_ref[...]
    idx = jnp.round(x)
    idx = jnp.clip(idx, 0, NUM_LEVELS - 1).astype(jnp.int32)
    c_of_col = lax.broadcasted_iota(jnp.int32, x.shape, 1) % cw_ref.shape[0]
    idx_ref[...] = idx + c_of_col * NUM_LEVELS
    bt_ref[...] = lw_ref[...][None, :, :] * cw_ref[...][:, None, :]


def _ngram_kernel(smp_ref, cen_ref, out_ref):
    x = smp_ref[...]
    s = x.shape[1]
    n = N_GRAM_SIZE
    r3 = jnp.roll(x, 3, axis=-1)[:, 0 : s - (n - 1), :]
    r2 = jnp.roll(x, 2, axis=-1)[:, 1 : s - 2, :]
    r1 = jnp.roll(x, 1, axis=-1)[:, 2 : s - 1, :]
    x0 = x[:, 3:s, :]
    prod = (r3 * r2) * (r1 * x0)
    shv = jnp.sum(prod, axis=1)
    hv = jnp.where(shv > 0, 1.0, -1.0).astype(jnp.float32)
    out_ref[...] = lax.dot_general(
        hv,
        cen_ref[...],
        (((1,), (1,)), ((), ())),
        preferred_element_type=jnp.float32,
        precision=lax.Precision.HIGHEST,
    )


def kernel(input, level_weight, channel_weight, centroid_weight):
    B, S, C = input.shape
    L, D = level_weight.shape
    NCLS = centroid_weight.shape[0]
    flat = B * S * C
    pairs = B * S

    y = (input - LOW) / (HIGH - LOW) * (NUM_LEVELS - 1)
    inp2d = y.reshape(flat // 128, 128)
    idx2d, bt3 = pl.pallas_call(
        _prep_kernel,
        out_shape=[
            jax.ShapeDtypeStruct((flat // 128, 128), jnp.int32),
            jax.ShapeDtypeStruct((C, L, D), jnp.float32),
        ],
    )(inp2d, level_weight, channel_weight)
    cidx = idx2d.reshape(flat)
    btable = bt3.reshape(C * L, D)

    ppw = pairs // _NW
    pch = 8
    nchunk = ppw // pch

    mesh = plsc.VectorSubcoreMesh(core_axis_name="c", subcore_axis_name="s")

    @functools.partial(
        pl.kernel,
        mesh=mesh,
        out_type=jax.ShapeDtypeStruct((pairs, D), jnp.float32),
        scratch_types=[
            pltpu.VMEM((pch * C,), jnp.int32),
            pltpu.VMEM((pch * C, D), jnp.float32),
            pltpu.VMEM((pch, D), jnp.float32),
            pltpu.SemaphoreType.DMA,
        ],
    )
    def _gather_sum(bt_hbm, cidx_hbm, out_hbm, idx_v, rows_v, acc_v, sem):
        wid = lax.axis_index("s") * _SC_CORES + lax.axis_index("c")
        base_pair = wid * ppw

        def chunk_body(ci, carry):
            pbase = base_pair + ci * pch
            ibase = pl.multiple_of(pbase * C, 8)
            pltpu.sync_copy(cidx_hbm.at[pl.ds(ibase, pch * C)], idx_v)
            pltpu.async_copy(bt_hbm.at[idx_v], rows_v, sem).wait()
            for p in range(pch):

                def vec_body(j, c2):
                    o = pl.ds(j * 16, 16)
                    acc = (rows_v[C * p, o] + rows_v[C * p + 1, o]) + (
                        rows_v[C * p + 2, o] + rows_v[C * p + 3, o]
                    )
                    acc_v[p, o] = acc
                    return c2

                lax.fori_loop(0, D // 16, vec_body, 0)
            pltpu.sync_copy(acc_v, out_hbm.at[pl.ds(pbase, pch)])
            return carry

        lax.fori_loop(0, nchunk, chunk_body, 0)

    samples = _gather_sum(btable, cidx)

    bt_batch = 8
    out = pl.pallas_call(
        _ngram_kernel,
        grid=(B // bt_batch,),
        in_specs=[
            pl.BlockSpec((bt_batch, S, D), lambda i: (i, 0, 0)),
            pl.BlockSpec((NCLS, D), lambda i: (0, 0)),
        ],
        out_specs=pl.BlockSpec((bt_batch, NCLS), lambda i: (i, 0)),
        out_shape=jax.ShapeDtypeStruct((B, NCLS), jnp.float32),
    )(samples.reshape(B, S, D), centroid_weight)
    return out


# speedup vs baseline: 2.7161x; 1.3692x over previous
import functools

import jax
import jax.numpy as jnp
from jax import lax
from jax.experimental import pallas as pl
from jax.experimental.pallas import tpu as pltpu
from jax.experimental.pallas import tpu_sc as plsc

NUM_LEVELS = 201
N_GRAM_SIZE = 4
LOW, HIGH = -100.0, 100.0

_SC_CORES = 2
_SC_SUBCORES = 16
_NW = _SC_CORES * _SC_SUBCORES


def _prep_kernel(inp_ref, lw_ref, cw_ref, idx_ref, btl_ref, bth_ref):
    x = inp_ref[...]
    idx = jnp.round(x)
    idx = jnp.clip(idx, 0, NUM_LEVELS - 1).astype(jnp.int32)
    c_of_col = lax.broadcasted_iota(jnp.int32, x.shape, 1) % cw_ref.shape[0]
    idx_ref[...] = idx + c_of_col * NUM_LEVELS
    half = btl_ref.shape[-1]
    bound = lw_ref[...][None, :, :] * cw_ref[...][:, None, :]
    btl_ref[...] = bound[..., :half]
    bth_ref[...] = bound[..., half:]


def _ngram_kernel(lo_ref, hi_ref, cen_ref, out_ref):
    x = jnp.concatenate([lo_ref[...], hi_ref[...]], axis=-1)
    s = x.shape[1]
    n = N_GRAM_SIZE
    r3 = jnp.roll(x, 3, axis=-1)[:, 0 : s - (n - 1), :]
    r2 = jnp.roll(x, 2, axis=-1)[:, 1 : s - 2, :]
    r1 = jnp.roll(x, 1, axis=-1)[:, 2 : s - 1, :]
    x0 = x[:, 3:s, :]
    prod = (r3 * r2) * (r1 * x0)
    shv = jnp.sum(prod, axis=1)
    hv = jnp.where(shv > 0, 1.0, -1.0).astype(jnp.float32)
    out_ref[...] = lax.dot_general(
        hv,
        cen_ref[...],
        (((1,), (1,)), ((), ())),
        preferred_element_type=jnp.float32,
        precision=lax.Precision.HIGHEST,
    )


def kernel(input, level_weight, channel_weight, centroid_weight):
    B, S, C = input.shape
    L, D = level_weight.shape
    NCLS = centroid_weight.shape[0]
    flat = B * S * C
    pairs = B * S
    Dh = D // 2

    y = (input - LOW) / (HIGH - LOW) * (NUM_LEVELS - 1)
    inp2d = y.reshape(flat // 128, 128)
    idx2d, btl3, bth3 = pl.pallas_call(
        _prep_kernel,
        out_shape=[
            jax.ShapeDtypeStruct((flat // 128, 128), jnp.int32),
            jax.ShapeDtypeStruct((C, L, Dh), jnp.float32),
            jax.ShapeDtypeStruct((C, L, Dh), jnp.float32),
        ],
    )(inp2d, level_weight, channel_weight)
    cidx = idx2d.reshape(flat)
    btl = btl3.reshape(C * L, Dh)
    bth = bth3.reshape(C * L, Dh)

    ppw = pairs // _NW
    pch = 8
    nchunk = ppw // pch
    nrows = pch * C

    mesh = plsc.VectorSubcoreMesh(core_axis_name="c", subcore_axis_name="s")

    @functools.partial(
        pl.kernel,
        mesh=mesh,
        out_type=[
            jax.ShapeDtypeStruct((pairs, Dh), jnp.float32),
            jax.ShapeDtypeStruct((pairs, Dh), jnp.float32),
        ],
        scratch_types=[
            pltpu.VMEM((2, nrows), jnp.int32),
            pltpu.VMEM((nrows, Dh), jnp.float32),
            pltpu.VMEM((nrows, Dh), jnp.float32),
            pltpu.VMEM((2, pch, Dh), jnp.float32),
            pltpu.SemaphoreType.DMA,
            pltpu.SemaphoreType.DMA,
            pltpu.SemaphoreType.DMA,
            pltpu.SemaphoreType.DMA,
        ],
    )
    def _gather_sum(
        btl_hbm,
        bth_hbm,
        cidx_hbm,
        out0_hbm,
        out1_hbm,
        idx_v,
        rows0_v,
        rows1_v,
        acc_v,
        gsem0,
        gsem1,
        osem0,
        osem1,
    ):
        wid = lax.axis_index("s") * _SC_CORES + lax.axis_index("c")
        base_pair = wid * ppw
        bt = (btl_hbm, bth_hbm)
        outs = (out0_hbm, out1_hbm)
        rows = (rows0_v, rows1_v)
        gsems = (gsem0, gsem1)
        osems = (osem0, osem1)

        def load_idx(ci, slot):
            off = pl.multiple_of((base_pair + ci * pch) * C, 32)
            pltpu.sync_copy(cidx_hbm.at[pl.ds(off, nrows)], idx_v.at[slot])

        def start_gather(half, slot):
            return pltpu.async_copy(
                bt[half].at[idx_v.at[slot]], rows[half], gsems[half]
            )

        def compute(half):
            for p in range(pch):

                def vec_body(j, c2):
                    o = pl.ds(j * 16, 16)
                    r = rows[half]
                    acc = (r[C * p, o] + r[C * p + 1, o]) + (
                        r[C * p + 2, o] + r[C * p + 3, o]
                    )
                    acc_v[half, p, o] = acc
                    return c2

                lax.fori_loop(0, Dh // 16, vec_body, 0)

        def start_out(ci, half):
            pbase = base_pair + ci * pch
            return pltpu.async_copy(
                acc_v.at[half], outs[half].at[pl.ds(pbase, pch)], osems[half]
            )

        def wait_gather(half, slot):
            pltpu.make_async_copy(
                bt[half].at[idx_v.at[slot]], rows[half], gsems[half]
            ).wait()

        def wait_out(half):
            pltpu.make_async_copy(
                acc_v.at[half], outs[half].at[pl.ds(0, pch)], osems[half]
            ).wait()

        load_idx(0, 0)
        start_gather(0, 0)
        load_idx(1, 1)
        wait_gather(0, 0)
        start_gather(1, 0)
        compute(0)
        start_out(0, 0)
        wait_gather(1, 0)
        start_gather(0, 1)
        compute(1)
        start_out(0, 1)

        def chunk_pair_body(c2i, carry):
            for par in range(2):
                c = 2 * c2i + 1 + par
                slot = (1 + par) % 2

                @pl.when(c < nchunk)
                def _do():
                    @pl.when(c + 1 < nchunk)
                    def _li():
                        load_idx(c + 1, 1 - slot)

                    wait_gather(0, slot)
                    start_gather(1, slot)
                    wait_out(0)
                    compute(0)
                    start_out(c, 0)
                    wait_gather(1, slot)

                    @pl.when(c + 1 < nchunk)
                    def _g0():
                        start_gather(0, 1 - slot)

                    wait_out(1)
                    compute(1)
                    start_out(c, 1)

            return carry

        lax.fori_loop(0, (nchunk - 1 + 1) // 2, chunk_pair_body, 0)

        wait_out(0)
        wait_out(1)

    samples_lo, samples_hi = _gather_sum(btl, bth, cidx)

    bt_batch = 8
    out = pl.pallas_call(
        _ngram_kernel,
        grid=(B // bt_batch,),
        in_specs=[
            pl.BlockSpec((bt_batch, S, Dh), lambda i: (i, 0, 0)),
            pl.BlockSpec((bt_batch, S, Dh), lambda i: (i, 0, 0)),
            pl.BlockSpec((NCLS, D), lambda i: (0, 0)),
        ],
        out_specs=pl.BlockSpec((bt_batch, NCLS), lambda i: (i, 0)),
        out_shape=jax.ShapeDtypeStruct((B, NCLS), jnp.float32),
    )(
        samples_lo.reshape(B, S, Dh),
        samples_hi.reshape(B, S, Dh),
        centroid_weight,
    )
    return out
